# baseline (speedup 1.0000x reference)
import functools
import math

import jax
import jax.numpy as jnp
from jax import lax
from jax.experimental import pallas as pl
from jax.experimental.pallas import tpu as pltpu
from jax.experimental.pallas import tpu_sc as plsc

_VOCAB = 100000
_EMB = 128
_SCALE = math.sqrt(float(_EMB))

_B = 4096 * 200
_NW = 32
_BPW = _B // _NW
_C = 128
_NCHUNK = _BPW // _C
_NBUF = 5
_GA = 3

_mesh = plsc.VectorSubcoreMesh(core_axis_name="c", subcore_axis_name="s")


@functools.partial(
    pl.kernel,
    mesh=_mesh,
    out_type=jax.ShapeDtypeStruct((_B, _EMB), jnp.float32),
    scratch_types=[
        pltpu.VMEM((_NCHUNK, _C), jnp.int32),
        pltpu.VMEM((_NBUF, _C, _EMB), jnp.float32),
        pltpu.SemaphoreType.DMA,
        pltpu.SemaphoreType.DMA,
    ],
)
def _gather(tokens_hbm, table_hbm, out_hbm, idx_v, rows_v, gsem, ssem):
    cid = lax.axis_index("c")
    sid = lax.axis_index("s")
    wid = sid * 2 + cid
    base = wid * _BPW

    pltpu.sync_copy(tokens_hbm.at[wid], idx_v)

    def g_copy(g, b):
        return pltpu.make_async_copy(
            table_hbm.at[idx_v.at[g]], rows_v.at[b], gsem
        )

    _H = _C // 2

    def s_copy(g, b, h):
        return pltpu.make_async_copy(
            rows_v.at[b, pl.ds(h * _H, _H)],
            out_hbm.at[pl.ds(base + g * _C + h * _H, _H)],
            ssem,
        )

    def scale_half(b, h):
        def sbody(i, carry):
            r0 = h * _H + i * 8
            for dr in range(8):
                for c in range(_EMB // 16):
                    sl = pl.ds(c * 16, 16)
                    rows_v[b, r0 + dr, sl] = rows_v[b, r0 + dr, sl] * _SCALE
            return carry

        lax.fori_loop(0, _H // 8, sbody, 0)

    def chunk(g, b, wait_s, issue_g):
        g_copy(g, b).wait()
        scale_half(b, 0)
        s_copy(g, b, 0).start()
        if wait_s:
            bp = (b - (_NBUF - _GA)) % _NBUF
            gp = g - (_NBUF - _GA)
            s_copy(gp, bp, 0).wait()
            s_copy(gp, bp, 1).wait()
        if issue_g:
            g_copy(g + _GA, (b + _GA) % _NBUF).start()
        scale_half(b, 1)
        s_copy(g, b, 1).start()

    for g in range(_GA):
        g_copy(g, g).start()

    for b in range(_NBUF):
        chunk(b, b, wait_s=(b >= _NBUF - _GA), issue_g=True)

    def body(i, carry):
        g0 = i * _NBUF
        for b in range(_NBUF):
            chunk(g0 + b, b, wait_s=True, issue_g=True)
        return carry

    lax.fori_loop(1, _NCHUNK // _NBUF - 1, body, 0)

    g0 = _NCHUNK - _NBUF
    for b in range(_NBUF):
        chunk(g0 + b, b, wait_s=True, issue_g=(b + _GA < _NBUF))

    for g in range(_NCHUNK - (_NBUF - _GA), _NCHUNK):
        s_copy(g, g % _NBUF, 0).wait()
        s_copy(g, g % _NBUF, 1).wait()


def kernel(tokens, table):
    tok = tokens.reshape(_NW, _NCHUNK, _C).astype(jnp.int32)
    out = _gather(tok, table)
    return out.reshape(tokens.shape[0], tokens.shape[1], _EMB)
